# baseline (speedup 1.0000x reference)
import functools

import jax
import jax.numpy as jnp
from jax import lax
from jax.experimental import pallas as pl
from jax.experimental.pallas import tpu as pltpu
from jax.experimental.pallas import tpu_sc as plsc

VOCAB = 100000
D = 768
MAX_POS = 8192
BATCH = 4
SEQ = 4096

L = 16
ND = D // L
NC = 2
NS = 16
NW = NC * NS
TOKENS = BATCH * SEQ
TPW = TOKENS // NW
C = 64
NCHUNK = TPW // C
EPS = 1e-5


def _lane_sum(x):
    vals = [x[i] for i in range(L)]
    while len(vals) > 1:
        vals = [a + b for a, b in zip(vals[0::2], vals[1::2])]
    return vals[0]


def _rsqrt_scalar(v):
    i = lax.bitcast_convert_type(v, jnp.int32)
    i = jnp.int32(0x5F3759DF) - (i >> 1)
    y = lax.bitcast_convert_type(i, jnp.float32)
    for _ in range(4):
        y = y * (1.5 - 0.5 * v * y * y)
    return y


@functools.partial(
    pl.kernel,
    out_type=jax.ShapeDtypeStruct((TOKENS, D), jnp.float32),
    mesh=plsc.VectorSubcoreMesh(core_axis_name="c", subcore_axis_name="s"),
    scratch_types=[
        pltpu.VMEM((C,), jnp.int32),
        pltpu.VMEM((C, D), jnp.float32),
        pltpu.VMEM((C, D), jnp.float32),
        pltpu.VMEM((D,), jnp.float32),
        pltpu.VMEM((D,), jnp.float32),
        pltpu.SemaphoreType.DMA,
    ],
)
def _emb_ln(ids_hbm, wtok_hbm, wpos_hbm, gamma_hbm, beta_hbm, out_hbm,
            idx_c, tok_v, pos_v, gam_v, bet_v, sem):
    wid = lax.axis_index("s") * NC + lax.axis_index("c")
    base = wid * TPW
    pos_base = lax.rem(base, SEQ)

    pltpu.sync_copy(gamma_hbm, gam_v)
    pltpu.sync_copy(beta_hbm, bet_v)

    def chunk_body(ci, carry):
        off = ci * C
        pltpu.sync_copy(ids_hbm.at[pl.ds(base + off, C)], idx_c)
        pltpu.async_copy(wtok_hbm.at[idx_c], tok_v, sem).wait()
        pltpu.sync_copy(wpos_hbm.at[pl.ds(pos_base + off, C)], pos_v)

        def tok_body(t, tc):
            s_acc = jnp.zeros((L,), jnp.float32)
            q_acc = jnp.zeros((L,), jnp.float32)
            for j in range(ND):
                sl = pl.ds(j * L, L)
                x = tok_v[t, sl] + pos_v[t, sl]
                tok_v[t, sl] = x
                s_acc = s_acc + x
                q_acc = q_acc + x * x
            mean_s = _lane_sum(s_acc) * (1.0 / D)
            msq_s = _lane_sum(q_acc) * (1.0 / D)
            var_s = msq_s - mean_s * mean_s
            mean = jnp.full((L,), mean_s, jnp.float32)
            rstd = jnp.full((L,), _rsqrt_scalar(var_s + EPS), jnp.float32)
            for j in range(ND):
                sl = pl.ds(j * L, L)
                x = tok_v[t, sl]
                tok_v[t, sl] = (x - mean) * rstd * gam_v[sl] + bet_v[sl]
            return tc

        lax.fori_loop(0, C, tok_body, 0)
        pltpu.sync_copy(tok_v, out_hbm.at[pl.ds(base + off, C)])
        return carry

    lax.fori_loop(0, NCHUNK, chunk_body, 0)


def kernel(input_ids, W_tok, W_pos, gamma, beta):
    ids = input_ids.astype(jnp.int32).reshape(-1)
    out = _emb_ln(ids, W_tok, W_pos, gamma, beta)
    return out.reshape(input_ids.shape[0], input_ids.shape[1], D)
